# baseline (speedup 1.0000x reference)
import functools

import jax
import jax.numpy as jnp
from jax import lax
from jax.experimental import pallas as pl
from jax.experimental.pallas import tpu as pltpu
from jax.experimental.pallas import tpu_sc as plsc

_K = 32
_NEG = -3.0e38


def _encode_body(x_ref, w_ref, b_ref, out_ref):
    acc = lax.dot_general(x_ref[...], w_ref[...], (((1,), (1,)), ((), ())),
                          preferred_element_type=jnp.float32)
    out_ref[...] = acc + b_ref[...]


def _select_body(pre_ref, hidden_ref, vals_ref, idx_ref, l0_ref, *, rb, h, nb):
    w = h // nb
    x = pre_ref[...]
    z = x.reshape(rb, w, nb)
    m = jnp.max(z, axis=1)
    colb = lax.broadcasted_iota(jnp.int32, (rb, nb), 1)
    lane_k = lax.broadcasted_iota(jnp.int32, (rb, _K), 1)

    def bucket_body(k, carry):
        mm, order = carry
        mx = jnp.max(mm, axis=1, keepdims=True)
        bi = jnp.min(jnp.where(mm == mx, colb, nb), axis=1, keepdims=True)
        order = jnp.where(lane_k == k, bi, order)
        mm = jnp.where(colb == bi, _NEG, mm)
        return mm, order

    _, order = lax.fori_loop(0, _K, bucket_body,
                             (m, jnp.zeros((rb, _K), jnp.int32)))

    order_b = jnp.broadcast_to(order[:, None, :], (rb, w, _K))
    cand = jnp.take_along_axis(z, order_b, axis=2,
                               mode="fill")
    ccol = (lax.broadcasted_iota(jnp.int32, (rb, w, _K), 1) * nb + order_b)
    pos = (lax.broadcasted_iota(jnp.int32, (rb, w, _K), 1) * _K
           + lax.broadcasted_iota(jnp.int32, (rb, w, _K), 2))
    npos = w * _K

    def cand_body(k, carry):
        xx, vals, idxs = carry
        mx = jnp.max(xx, axis=(1, 2), keepdims=True)
        is_mx = xx == mx
        ci = jnp.min(jnp.where(is_mx, pos, npos), axis=(1, 2), keepdims=True)
        sel = pos == ci
        gi = jnp.min(jnp.where(sel, ccol, h), axis=(1, 2))
        vals = jnp.where(lane_k == k, mx[:, :, 0], vals)
        idxs = jnp.where(lane_k == k, gi[:, None], idxs)
        xx = jnp.where(sel, _NEG, xx)
        return xx, vals, idxs

    init = (cand,
            jnp.zeros((rb, _K), jnp.float32),
            jnp.zeros((rb, _K), jnp.int32))
    _, vals, idxs = lax.fori_loop(0, _K, cand_body, init)

    t32 = jnp.min(vals, axis=1, keepdims=True)
    hidden_ref[...] = jnp.where(x >= t32, jnp.maximum(x, 0.0), 0.0)
    relu_vals = jnp.maximum(vals, 0.0)
    vals_ref[...] = jnp.repeat(relu_vals, 16, axis=1)
    idx_ref[...] = idxs

    @pl.when(pl.program_id(0) == 0)
    def _():
        l0_ref[...] = jnp.zeros_like(l0_ref)

    l0_ref[...] += jnp.sum((vals > 0.0).astype(jnp.float32), keepdims=True)


def _sc_decode_body(wdec_ref, idx_ref, vals_ref, out_ref,
                    idx_v, vals_v, rows_v, stage_v, sem, *, tpw, d_out):
    lanes = 16
    wid = lax.axis_index("s") * 2 + lax.axis_index("c")
    base = wid * tpw

    def token_body(t, carry):
        tok = base + t
        pltpu.sync_copy(idx_ref.at[tok], idx_v)
        pltpu.sync_copy(vals_ref.at[tok], vals_v)
        pltpu.async_copy(wdec_ref.at[idx_v], rows_v, sem).wait()

        def d_body(d, c):
            sl = pl.ds(d * lanes, lanes)
            acc = vals_v[pl.ds(0, lanes)] * rows_v[0, sl]
            for k in range(1, _K):
                acc = acc + vals_v[pl.ds(k * lanes, lanes)] * rows_v[k, sl]
            stage_v[sl] = acc
            return c

        lax.fori_loop(0, d_out // lanes, d_body, 0)
        pltpu.sync_copy(stage_v, out_ref.at[tok])
        return carry

    lax.fori_loop(0, tpw, token_body, 0)


def _skip_body(x_ref, w_ref, b_ref, sp_ref, y_ref, pred_ref, loss_ref):
    pred = lax.dot_general(x_ref[...], w_ref[...], (((1,), (1,)), ((), ())),
                           preferred_element_type=jnp.float32)
    pred = pred + b_ref[...] + sp_ref[...]
    pred_ref[...] = pred
    dif = pred - y_ref[...]

    @pl.when(pl.program_id(0) == 0)
    def _():
        loss_ref[...] = jnp.zeros_like(loss_ref)

    loss_ref[...] += jnp.sum(dif * dif, keepdims=True)


def kernel(mlp_input, mlp_output, W_enc, b_enc, W_dec, b_dec, W_skip, b_skip):
    n, d_in = mlp_input.shape
    h = W_enc.shape[0]
    d_out = W_dec.shape[0]
    f32 = jnp.float32

    hb = 512
    pre = pl.pallas_call(
        _encode_body,
        grid=(h // hb,),
        in_specs=[
            pl.BlockSpec((n, d_in), lambda j: (0, 0)),
            pl.BlockSpec((hb, d_in), lambda j: (j, 0)),
            pl.BlockSpec((1, hb), lambda j: (0, j)),
        ],
        out_specs=pl.BlockSpec((n, hb), lambda j: (0, j)),
        out_shape=jax.ShapeDtypeStruct((n, h), f32),
    )(mlp_input, W_enc, b_enc.reshape(1, h))

    rb = 128
    hidden, vals, idxs, l0_sum = pl.pallas_call(
        functools.partial(_select_body, rb=rb, h=h, nb=128),
        grid=(n // rb,),
        in_specs=[pl.BlockSpec((rb, h), lambda i: (i, 0))],
        out_specs=[
            pl.BlockSpec((rb, h), lambda i: (i, 0)),
            pl.BlockSpec((rb, _K * 16), lambda i: (i, 0)),
            pl.BlockSpec((rb, _K), lambda i: (i, 0)),
            pl.BlockSpec((1, 1), lambda i: (0, 0)),
        ],
        out_shape=[
            jax.ShapeDtypeStruct((n, h), f32),
            jax.ShapeDtypeStruct((n, _K * 16), f32),
            jax.ShapeDtypeStruct((n, _K), jnp.int32),
            jax.ShapeDtypeStruct((1, 1), f32),
        ],
    )(pre)

    nw = 32
    tpw = n // nw
    wdec_t = W_dec.T
    mesh = plsc.VectorSubcoreMesh(core_axis_name="c", subcore_axis_name="s")
    sc_decode = functools.partial(
        pl.kernel,
        mesh=mesh,
        out_type=jax.ShapeDtypeStruct((n, d_out), f32),
        scratch_types=[
            pltpu.VMEM((_K,), jnp.int32),
            pltpu.VMEM((_K * 16,), f32),
            pltpu.VMEM((_K, d_out), f32),
            pltpu.VMEM((d_out,), f32),
            pltpu.SemaphoreType.DMA,
        ],
    )(functools.partial(_sc_decode_body, tpw=tpw, d_out=d_out))
    sparse_out = sc_decode(wdec_t, idxs, vals)

    bias = (b_dec + b_skip).reshape(1, d_out)
    sb = 512
    predicted, loss_sum = pl.pallas_call(
        _skip_body,
        grid=(n // sb,),
        in_specs=[
            pl.BlockSpec((sb, d_in), lambda i: (i, 0)),
            pl.BlockSpec((d_out, d_in), lambda i: (0, 0)),
            pl.BlockSpec((1, d_out), lambda i: (0, 0)),
            pl.BlockSpec((sb, d_out), lambda i: (i, 0)),
            pl.BlockSpec((sb, d_out), lambda i: (i, 0)),
        ],
        out_specs=[
            pl.BlockSpec((sb, d_out), lambda i: (i, 0)),
            pl.BlockSpec((1, 1), lambda i: (0, 0)),
        ],
        out_shape=[
            jax.ShapeDtypeStruct((n, d_out), f32),
            jax.ShapeDtypeStruct((1, 1), f32),
        ],
    )(mlp_input, W_skip, bias, sparse_out, mlp_output)

    loss = (loss_sum[0, 0] / (n * d_out)).astype(f32)
    l0 = (l0_sum[0, 0] / n).astype(f32)
    sparsity_loss = jnp.zeros((), f32)
    return (predicted, hidden, loss, loss, sparsity_loss, l0)


# speedup vs baseline: 2.2148x; 2.2148x over previous
import functools

import jax
import jax.numpy as jnp
from jax import lax
from jax.experimental import pallas as pl
from jax.experimental.pallas import tpu as pltpu
from jax.experimental.pallas import tpu_sc as plsc

_K = 32
_NEG = -3.0e38


def _encode_body(x_ref, w_ref, b_ref, out_ref):
    acc = lax.dot_general(x_ref[...], w_ref[...], (((1,), (1,)), ((), ())),
                          preferred_element_type=jnp.float32)
    out_ref[...] = acc + b_ref[...]


def _select_body(pre_ref, hidden_ref, vals_ref, idx_ref, l0_ref, *, rb, h, nb):
    w = h // nb
    x = pre_ref[...]
    z = x.reshape(rb, w, nb)
    m = jnp.max(z, axis=1)
    colb = lax.broadcasted_iota(jnp.int32, (rb, nb), 1)
    lane_k = lax.broadcasted_iota(jnp.int32, (rb, _K), 1)

    def bucket_body(k, carry):
        mm, order = carry
        mx = jnp.max(mm, axis=1, keepdims=True)
        bi = jnp.min(jnp.where(mm == mx, colb, nb), axis=1, keepdims=True)
        order = jnp.where(lane_k == k, bi, order)
        mm = jnp.where(colb == bi, _NEG, mm)
        return mm, order

    _, order = lax.fori_loop(0, _K, bucket_body,
                             (m, jnp.zeros((rb, _K), jnp.int32)))

    order_b = jnp.broadcast_to(order[:, None, :], (rb, w, _K))
    cand = jnp.take_along_axis(z, order_b, axis=2,
                               mode="fill")
    ccol = (lax.broadcasted_iota(jnp.int32, (rb, w, _K), 1) * nb + order_b)
    nc = w * _K
    cand2 = cand.reshape(rb, nc)
    ccol2 = ccol.reshape(rb, nc)
    colc = lax.broadcasted_iota(jnp.int32, (rb, nc), 1)

    def cand_body(k, carry):
        xx, vals, idxs = carry
        mx = jnp.max(xx, axis=1, keepdims=True)
        ci = jnp.min(jnp.where(xx == mx, colc, nc), axis=1, keepdims=True)
        sel = colc == ci
        gi = jnp.min(jnp.where(sel, ccol2, h), axis=1, keepdims=True)
        vals = jnp.where(lane_k == k, mx, vals)
        idxs = jnp.where(lane_k == k, gi, idxs)
        xx = jnp.where(sel, _NEG, xx)
        return xx, vals, idxs

    init = (cand2,
            jnp.zeros((rb, _K), jnp.float32),
            jnp.zeros((rb, _K), jnp.int32))
    _, vals, idxs = lax.fori_loop(0, _K, cand_body, init)

    t32 = jnp.min(vals, axis=1, keepdims=True)
    hidden_ref[...] = jnp.where(x >= t32, jnp.maximum(x, 0.0), 0.0)
    relu_vals = jnp.maximum(vals, 0.0)
    vals_ref[...] = jnp.repeat(relu_vals, 16, axis=1)
    idx_ref[...] = idxs

    @pl.when(pl.program_id(0) == 0)
    def _():
        l0_ref[...] = jnp.zeros_like(l0_ref)

    l0_ref[...] += jnp.sum((vals > 0.0).astype(jnp.float32), keepdims=True)


def _sc_decode_body(wdec_ref, idx_ref, vals_ref, out_ref,
                    idx_v, vals_v, rows_v, stage_v, sem, *, tpw, d_out):
    lanes = 16
    wid = lax.axis_index("s") * 2 + lax.axis_index("c")
    base = wid * tpw

    def token_body(t, carry):
        tok = base + t
        pltpu.sync_copy(idx_ref.at[tok], idx_v)
        pltpu.sync_copy(vals_ref.at[tok], vals_v)
        pltpu.async_copy(wdec_ref.at[idx_v], rows_v, sem).wait()

        def d_body(d, c):
            sl = pl.ds(d * lanes, lanes)
            acc = vals_v[pl.ds(0, lanes)] * rows_v[0, sl]
            for k in range(1, _K):
                acc = acc + vals_v[pl.ds(k * lanes, lanes)] * rows_v[k, sl]
            stage_v[sl] = acc
            return c

        lax.fori_loop(0, d_out // lanes, d_body, 0)
        pltpu.sync_copy(stage_v, out_ref.at[tok])
        return carry

    lax.fori_loop(0, tpw, token_body, 0)


def _skip_body(x_ref, w_ref, b_ref, sp_ref, y_ref, pred_ref, loss_ref):
    pred = lax.dot_general(x_ref[...], w_ref[...], (((1,), (1,)), ((), ())),
                           preferred_element_type=jnp.float32)
    pred = pred + b_ref[...] + sp_ref[...]
    pred_ref[...] = pred
    dif = pred - y_ref[...]

    @pl.when(pl.program_id(0) == 0)
    def _():
        loss_ref[...] = jnp.zeros_like(loss_ref)

    loss_ref[...] += jnp.sum(dif * dif, keepdims=True)


def kernel(mlp_input, mlp_output, W_enc, b_enc, W_dec, b_dec, W_skip, b_skip):
    n, d_in = mlp_input.shape
    h = W_enc.shape[0]
    d_out = W_dec.shape[0]
    f32 = jnp.float32

    hb = 512
    pre = pl.pallas_call(
        _encode_body,
        grid=(h // hb,),
        in_specs=[
            pl.BlockSpec((n, d_in), lambda j: (0, 0)),
            pl.BlockSpec((hb, d_in), lambda j: (j, 0)),
            pl.BlockSpec((1, hb), lambda j: (0, j)),
        ],
        out_specs=pl.BlockSpec((n, hb), lambda j: (0, j)),
        out_shape=jax.ShapeDtypeStruct((n, h), f32),
    )(mlp_input, W_enc, b_enc.reshape(1, h))

    rb = 128
    hidden, vals, idxs, l0_sum = pl.pallas_call(
        functools.partial(_select_body, rb=rb, h=h, nb=128),
        grid=(n // rb,),
        in_specs=[pl.BlockSpec((rb, h), lambda i: (i, 0))],
        out_specs=[
            pl.BlockSpec((rb, h), lambda i: (i, 0)),
            pl.BlockSpec((rb, _K * 16), lambda i: (i, 0)),
            pl.BlockSpec((rb, _K), lambda i: (i, 0)),
            pl.BlockSpec((1, 1), lambda i: (0, 0)),
        ],
        out_shape=[
            jax.ShapeDtypeStruct((n, h), f32),
            jax.ShapeDtypeStruct((n, _K * 16), f32),
            jax.ShapeDtypeStruct((n, _K), jnp.int32),
            jax.ShapeDtypeStruct((1, 1), f32),
        ],
    )(pre)

    nw = 32
    tpw = n // nw
    wdec_t = W_dec.T
    mesh = plsc.VectorSubcoreMesh(core_axis_name="c", subcore_axis_name="s")
    sc_decode = functools.partial(
        pl.kernel,
        mesh=mesh,
        out_type=jax.ShapeDtypeStruct((n, d_out), f32),
        scratch_types=[
            pltpu.VMEM((_K,), jnp.int32),
            pltpu.VMEM((_K * 16,), f32),
            pltpu.VMEM((_K, d_out), f32),
            pltpu.VMEM((d_out,), f32),
            pltpu.SemaphoreType.DMA,
        ],
    )(functools.partial(_sc_decode_body, tpw=tpw, d_out=d_out))
    sparse_out = sc_decode(wdec_t, idxs, vals)

    bias = (b_dec + b_skip).reshape(1, d_out)
    sb = 512
    predicted, loss_sum = pl.pallas_call(
        _skip_body,
        grid=(n // sb,),
        in_specs=[
            pl.BlockSpec((sb, d_in), lambda i: (i, 0)),
            pl.BlockSpec((d_out, d_in), lambda i: (0, 0)),
            pl.BlockSpec((1, d_out), lambda i: (0, 0)),
            pl.BlockSpec((sb, d_out), lambda i: (i, 0)),
            pl.BlockSpec((sb, d_out), lambda i: (i, 0)),
        ],
        out_specs=[
            pl.BlockSpec((sb, d_out), lambda i: (i, 0)),
            pl.BlockSpec((1, 1), lambda i: (0, 0)),
        ],
        out_shape=[
            jax.ShapeDtypeStruct((n, d_out), f32),
            jax.ShapeDtypeStruct((1, 1), f32),
        ],
    )(mlp_input, W_skip, bias, sparse_out, mlp_output)

    loss = (loss_sum[0, 0] / (n * d_out)).astype(f32)
    l0 = (l0_sum[0, 0] / n).astype(f32)
    sparsity_loss = jnp.zeros((), f32)
    return (predicted, hidden, loss, loss, sparsity_loss, l0)


# speedup vs baseline: 2.5081x; 1.1324x over previous
import functools

import jax
import jax.numpy as jnp
from jax import lax
from jax.experimental import pallas as pl
from jax.experimental.pallas import tpu as pltpu
from jax.experimental.pallas import tpu_sc as plsc

_K = 32
_NEG = -3.0e38


def _encode_body(x_ref, w_ref, b_ref, out_ref):
    acc = lax.dot_general(x_ref[...], w_ref[...], (((1,), (1,)), ((), ())),
                          preferred_element_type=jnp.float32)
    out_ref[...] = acc + b_ref[...]


def _select_body(pre_ref, hidden_ref, vals_ref, idx_ref, l0_ref, *, rb, h, nb):
    w = h // nb
    x = pre_ref[...]
    z = x.reshape(rb, w, nb)
    m = jnp.max(z, axis=1)
    colb = lax.broadcasted_iota(jnp.int32, (rb, nb), 1)
    lane_k = lax.broadcasted_iota(jnp.int32, (rb, _K), 1)

    def bucket_body(k, carry):
        mm, order = carry
        mx = jnp.max(mm, axis=1, keepdims=True)
        bi = jnp.min(jnp.where(mm == mx, colb, nb), axis=1, keepdims=True)
        order = jnp.where(lane_k == k, bi, order)
        mm = jnp.where(colb == bi, _NEG, mm)
        return mm, order

    _, order = lax.fori_loop(0, _K, bucket_body,
                             (m, jnp.zeros((rb, _K), jnp.int32)))

    order_b = jnp.broadcast_to(order[:, None, :], (rb, w, _K))
    cand = jnp.take_along_axis(z, order_b, axis=2,
                               mode="fill")
    ccol = (lax.broadcasted_iota(jnp.int32, (rb, w, _K), 1) * nb + order_b)
    nc = w * _K
    cand2 = cand.reshape(rb, nc)
    ccol2 = ccol.reshape(rb, nc)
    colc = lax.broadcasted_iota(jnp.int32, (rb, nc), 1)

    def cand_body(k, carry):
        xx, vals, idxs = carry
        mx = jnp.max(xx, axis=1, keepdims=True)
        ci = jnp.min(jnp.where(xx == mx, colc, nc), axis=1, keepdims=True)
        sel = colc == ci
        gi = jnp.min(jnp.where(sel, ccol2, h), axis=1, keepdims=True)
        vals = jnp.where(lane_k == k, mx, vals)
        idxs = jnp.where(lane_k == k, gi, idxs)
        xx = jnp.where(sel, _NEG, xx)
        return xx, vals, idxs

    init = (cand2,
            jnp.zeros((rb, _K), jnp.float32),
            jnp.zeros((rb, _K), jnp.int32))
    _, vals, idxs = lax.fori_loop(0, _K, cand_body, init)

    t32 = jnp.min(vals, axis=1, keepdims=True)
    hidden_ref[...] = jnp.where(x >= t32, jnp.maximum(x, 0.0), 0.0)
    relu_vals = jnp.maximum(vals, 0.0)
    vals_ref[...] = jnp.repeat(relu_vals, 16, axis=1)
    idx_ref[...] = idxs

    @pl.when(pl.program_id(0) == 0)
    def _():
        l0_ref[...] = jnp.zeros_like(l0_ref)

    l0_ref[...] += jnp.sum((vals > 0.0).astype(jnp.float32), keepdims=True)


def _sc_decode_body(wdec_ref, idx_ref, vals_ref, out_ref,
                    idx_all, vals_all, buf_a, buf_b, stage_v,
                    sem_a, sem_b, *, tpw, d_out):
    lanes = 16
    kh = _K // 2
    wid = lax.axis_index("s") * 2 + lax.axis_index("c")
    base = wid * tpw
    pltpu.sync_copy(idx_ref.at[pl.ds(base, tpw)], idx_all)
    pltpu.sync_copy(vals_ref.at[pl.ds(base, tpw)], vals_all)

    def gather(t, half, buf, sem):
        return pltpu.make_async_copy(
            wdec_ref.at[idx_all.at[t, pl.ds(half * kh, kh)]], buf, sem)

    def accumulate(buf, t, half):
        def d_body(d, c):
            sl = pl.ds(d * lanes, lanes)
            acc = vals_all[t, pl.ds(half * kh * lanes, lanes)] * buf[0, sl]
            for kk in range(1, kh):
                acc = acc + (vals_all[t, pl.ds((half * kh + kk) * lanes,
                                               lanes)] * buf[kk, sl])
            if half == 0:
                stage_v[sl] = acc
            else:
                plsc.addupdate(stage_v.at[pl.ds(d * lanes, lanes)], acc)
            return c

        lax.fori_loop(0, d_out // lanes, d_body, 0)

    gather(0, 0, buf_a, sem_a).start()

    def token_body(t, carry):
        gather(t, 1, buf_b, sem_b).start()
        gather(t, 0, buf_a, sem_a).wait()
        accumulate(buf_a, t, 0)

        @pl.when(t < tpw - 1)
        def _():
            gather(t + 1, 0, buf_a, sem_a).start()

        gather(t, 1, buf_b, sem_b).wait()
        accumulate(buf_b, t, 1)
        pltpu.sync_copy(stage_v, out_ref.at[base + t])
        return carry

    lax.fori_loop(0, tpw, token_body, 0)


def _skip_body(x_ref, w_ref, out_ref):
    out_ref[...] = lax.dot_general(x_ref[...], w_ref[...],
                                   (((1,), (1,)), ((), ())),
                                   preferred_element_type=jnp.float32)


def _combine_body(sk_ref, b_ref, sp_ref, y_ref, pred_ref, loss_ref):
    pred = sk_ref[...] + b_ref[...] + sp_ref[...]
    pred_ref[...] = pred
    dif = pred - y_ref[...]

    @pl.when(pl.program_id(0) == 0)
    def _():
        loss_ref[...] = jnp.zeros_like(loss_ref)

    loss_ref[...] += jnp.sum(dif * dif, keepdims=True)


def kernel(mlp_input, mlp_output, W_enc, b_enc, W_dec, b_dec, W_skip, b_skip):
    n, d_in = mlp_input.shape
    h = W_enc.shape[0]
    d_out = W_dec.shape[0]
    f32 = jnp.float32

    hb = 512
    pre = pl.pallas_call(
        _encode_body,
        grid=(h // hb,),
        in_specs=[
            pl.BlockSpec((n, d_in), lambda j: (0, 0)),
            pl.BlockSpec((hb, d_in), lambda j: (j, 0)),
            pl.BlockSpec((1, hb), lambda j: (0, j)),
        ],
        out_specs=pl.BlockSpec((n, hb), lambda j: (0, j)),
        out_shape=jax.ShapeDtypeStruct((n, h), f32),
    )(mlp_input, W_enc, b_enc.reshape(1, h))

    rb = 128
    hidden, vals, idxs, l0_sum = pl.pallas_call(
        functools.partial(_select_body, rb=rb, h=h, nb=128),
        grid=(n // rb,),
        in_specs=[pl.BlockSpec((rb, h), lambda i: (i, 0))],
        out_specs=[
            pl.BlockSpec((rb, h), lambda i: (i, 0)),
            pl.BlockSpec((rb, _K * 16), lambda i: (i, 0)),
            pl.BlockSpec((rb, _K), lambda i: (i, 0)),
            pl.BlockSpec((1, 1), lambda i: (0, 0)),
        ],
        out_shape=[
            jax.ShapeDtypeStruct((n, h), f32),
            jax.ShapeDtypeStruct((n, _K * 16), f32),
            jax.ShapeDtypeStruct((n, _K), jnp.int32),
            jax.ShapeDtypeStruct((1, 1), f32),
        ],
    )(pre)

    nw = 32
    tpw = n // nw
    wdec_t = W_dec.T
    mesh = plsc.VectorSubcoreMesh(core_axis_name="c", subcore_axis_name="s")
    sc_decode = functools.partial(
        pl.kernel,
        mesh=mesh,
        out_type=jax.ShapeDtypeStruct((n, d_out), f32),
        scratch_types=[
            pltpu.VMEM((tpw, _K), jnp.int32),
            pltpu.VMEM((tpw, _K * 16), f32),
            pltpu.VMEM((_K // 2, d_out), f32),
            pltpu.VMEM((_K // 2, d_out), f32),
            pltpu.VMEM((d_out,), f32),
            pltpu.SemaphoreType.DMA,
            pltpu.SemaphoreType.DMA,
        ],
    )(functools.partial(_sc_decode_body, tpw=tpw, d_out=d_out))
    sparse_out = sc_decode(wdec_t, idxs, vals)

    sb = 512
    skip_out = pl.pallas_call(
        _skip_body,
        grid=(n // sb,),
        in_specs=[
            pl.BlockSpec((sb, d_in), lambda i: (i, 0)),
            pl.BlockSpec((d_out, d_in), lambda i: (0, 0)),
        ],
        out_specs=pl.BlockSpec((sb, d_out), lambda i: (i, 0)),
        out_shape=jax.ShapeDtypeStruct((n, d_out), f32),
    )(mlp_input, W_skip)

    bias = (b_dec + b_skip).reshape(1, d_out)
    predicted, loss_sum = pl.pallas_call(
        _combine_body,
        grid=(n // sb,),
        in_specs=[
            pl.BlockSpec((sb, d_out), lambda i: (i, 0)),
            pl.BlockSpec((1, d_out), lambda i: (0, 0)),
            pl.BlockSpec((sb, d_out), lambda i: (i, 0)),
            pl.BlockSpec((sb, d_out), lambda i: (i, 0)),
        ],
        out_specs=[
            pl.BlockSpec((sb, d_out), lambda i: (i, 0)),
            pl.BlockSpec((1, 1), lambda i: (0, 0)),
        ],
        out_shape=[
            jax.ShapeDtypeStruct((n, d_out), f32),
            jax.ShapeDtypeStruct((1, 1), f32),
        ],
    )(skip_out, bias, sparse_out, mlp_output)

    loss = (loss_sum[0, 0] / (n * d_out)).astype(f32)
    l0 = (l0_sum[0, 0] / n).astype(f32)
    sparsity_loss = jnp.zeros((), f32)
    return (predicted, hidden, loss, loss, sparsity_loss, l0)
